# baseline (speedup 1.0000x reference)
import jax
import jax.numpy as jnp
from jax.experimental import pallas as pl
from jax.experimental.pallas import tpu as pltpu

E = 8
H = 1024
I_MOE = 512
I_SHARED = 1024
PAIRS = E // 2


def _dot_t(a, b):
    return jax.lax.dot_general(
        a, b, (((1,), (1,)), ((), ())), preferred_element_type=jnp.float32
    )


def _bf(v):
    return v.astype(jnp.bfloat16)


def _silu(x):
    return x * jax.nn.sigmoid(x)


def _moe_kernel(x_ref, gate_w_ref, gp_ref, up_ref, dp_ref,
                sg_ref, su_ref, sd_ref, seg_ref,
                out_ref, combine_ref, xb_ref):
    k = pl.program_id(0)
    t = out_ref.shape[0]
    e_iota = jax.lax.broadcasted_iota(jnp.int32, (t, E), 1)

    @pl.when(k == 0)
    def _router_and_shared():
        x = x_ref[...]
        xb_ref[...] = _bf(x)

        logits = _dot_t(x, gate_w_ref[...])
        m = jnp.max(logits, axis=-1, keepdims=True)
        p = jnp.exp(logits - m)
        p = p / jnp.sum(p, axis=-1, keepdims=True)

        w1 = jnp.max(p, axis=-1, keepdims=True)
        i1 = jnp.min(jnp.where(p == w1, e_iota, E), axis=-1, keepdims=True)
        m1 = e_iota == i1
        p2 = jnp.where(m1, -1.0, p)
        w2 = jnp.max(p2, axis=-1, keepdims=True)
        i2 = jnp.min(jnp.where(p2 == w2, e_iota, E), axis=-1, keepdims=True)
        m2 = e_iota == i2
        combine = jnp.where(m1, w1, 0.0) + jnp.where(m2, w2, 0.0)
        combine_ref[...] = combine / (w1 + w2)

        xbs = xb_ref[...]
        sg = _dot_t(xbs, _bf(sg_ref[...]))
        su = _dot_t(xbs, _bf(su_ref[...]))
        sh = _dot_t(_bf(_silu(sg) * su), _bf(sd_ref[...]))
        gv = jax.nn.sigmoid(_dot_t(x, seg_ref[...]))
        out_ref[...] = gv * sh

    xb = xb_ref[...]

    gp_pair = gp_ref[...].reshape(2 * I_MOE, H)
    up_pair = up_ref[...].reshape(2 * I_MOE, H)
    g12 = _dot_t(xb, _bf(gp_pair))
    u12 = _dot_t(xb, _bf(up_pair))
    act12 = _silu(g12) * u12

    combine = combine_ref[...]
    w_a = jnp.sum(jnp.where(e_iota == 2 * k, combine, 0.0),
                  axis=-1, keepdims=True)
    w_b = jnp.sum(jnp.where(e_iota == 2 * k + 1, combine, 0.0),
                  axis=-1, keepdims=True)
    half = jax.lax.broadcasted_iota(jnp.int32, (t, 2 * I_MOE), 1) < I_MOE
    act12 = _bf(act12 * jnp.where(half, w_a, w_b))

    contrib = _dot_t(act12[:, :I_MOE], _bf(dp_ref[0]))
    contrib = contrib + _dot_t(act12[:, I_MOE:], _bf(dp_ref[1]))

    out_ref[...] += contrib


def kernel(hidden_states, gate_w, gate_proj_w, up_proj_w, down_proj_w,
           shared_gate_w, shared_up_w, shared_down_w, shared_expert_gate_w):
    b, s, h = hidden_states.shape
    x = hidden_states.reshape(-1, h)
    t = x.shape[0]

    out = pl.pallas_call(
        _moe_kernel,
        grid=(PAIRS,),
        in_specs=[
            pl.BlockSpec((t, h), lambda i: (0, 0)),
            pl.BlockSpec((E, h), lambda i: (0, 0)),
            pl.BlockSpec((2, I_MOE, h), lambda i: (i, 0, 0)),
            pl.BlockSpec((2, I_MOE, h), lambda i: (i, 0, 0)),
            pl.BlockSpec((2, h, I_MOE), lambda i: (i, 0, 0)),
            pl.BlockSpec((I_SHARED, h), lambda i: (0, 0)),
            pl.BlockSpec((I_SHARED, h), lambda i: (0, 0)),
            pl.BlockSpec((h, I_SHARED), lambda i: (0, 0)),
            pl.BlockSpec((1, h), lambda i: (0, 0)),
        ],
        out_specs=pl.BlockSpec((t, h), lambda i: (0, 0)),
        out_shape=jax.ShapeDtypeStruct((t, h), jnp.float32),
        scratch_shapes=[
            pltpu.VMEM((t, E), jnp.float32),
            pltpu.VMEM((t, H), jnp.bfloat16),
        ],
        compiler_params=pltpu.CompilerParams(
            vmem_limit_bytes=64 * 1024 * 1024,
        ),
    )(x, gate_w, gate_proj_w, up_proj_w, down_proj_w,
      shared_gate_w, shared_up_w, shared_down_w, shared_expert_gate_w)

    return out.reshape(b, s, h)
